# baseline (speedup 1.0000x reference)
import functools

import jax
import jax.numpy as jnp
from jax import lax
from jax.experimental import pallas as pl
from jax.experimental.pallas import tpu as pltpu
from jax.experimental.pallas import tpu_sc as plsc

N_NODES = 10000
NPAD = 10240
D = 64
NW = 32
CHUNK = 128
ROWS_PT = NPAD // 16

_mesh = functools.partial(
    plsc.VectorSubcoreMesh, core_axis_name="c", subcore_axis_name="s")
_sc_params = pltpu.CompilerParams(use_tc_tiling_on_sc=False)


def _load_idx(real_hbm, pad_hbm, idx_v, wid, nch, nreal):
    npadr = NW * nch - nreal
    nlast = nch - npadr
    if npadr == 0:
        pltpu.sync_copy(real_hbm.at[pl.ds(wid * nch, nch)], idx_v)
        return

    @pl.when(wid < NW - 1)
    def _full():
        pltpu.sync_copy(real_hbm.at[pl.ds(wid * nch, nch)], idx_v)

    @pl.when(wid == NW - 1)
    def _split():
        pltpu.sync_copy(real_hbm.at[pl.ds(nreal - nlast, nlast)],
                        idx_v.at[pl.ds(0, nlast)])
        pltpu.sync_copy(pad_hbm, idx_v.at[pl.ds(nlast, npadr)])


def _make_deg_kernel(nch, nreal):
    @functools.partial(
        pl.kernel,
        mesh=_mesh(),
        compiler_params=_sc_params,
        out_type=jax.ShapeDtypeStruct((2, NPAD), jnp.float32),
        scratch_types=[
            pltpu.VMEM((nch, CHUNK), jnp.int32),
            pltpu.VMEM((CHUNK,), jnp.float32),
            pltpu.VMEM_SHARED((NPAD,), jnp.float32),
            [pltpu.SemaphoreType.DMA] * 4,
        ],
    )
    def deg_kernel(edge_hbm, dpad_hbm, init_hbm, out_hbm,
                   dst_v, ones_v, acc_sh, ssems):
        c = lax.axis_index("c")
        s = lax.axis_index("s")
        wid = s * 2 + c
        _load_idx(edge_hbm.at[1], dpad_hbm, dst_v, wid, nch, nreal)
        pltpu.sync_copy(init_hbm.at[pl.ds(0, CHUNK)], ones_v)
        pltpu.sync_copy(init_hbm.at[pl.ds(s * ROWS_PT, ROWS_PT)],
                        acc_sh.at[pl.ds(s * ROWS_PT, ROWS_PT)])
        plsc.subcore_barrier()

        def body(i, _):
            for b in range(4):
                j = i * 4 + b

                @pl.when(j >= 4)
                def _drain():
                    pltpu.make_async_copy(
                        ones_v, acc_sh.at[dst_v.at[j - 4]], ssems[b]).wait()

                pltpu.async_copy(ones_v, acc_sh.at[dst_v.at[j]],
                                 ssems[b], add=True)
            return _

        lax.fori_loop(0, nch // 4, body, None)
        for b in range(4):
            pltpu.make_async_copy(ones_v,
                                  acc_sh.at[dst_v.at[nch - 4 + b]],
                                  ssems[b]).wait()
        plsc.subcore_barrier()
        pltpu.sync_copy(acc_sh.at[pl.ds(s * ROWS_PT, ROWS_PT)],
                        out_hbm.at[c, pl.ds(s * ROWS_PT, ROWS_PT)])

    return deg_kernel


def _make_agg_kernel(nch, nreal):
    kb = 1
    nbuf = 8
    lookahead = 3
    nsup = nch // kb

    @functools.partial(
        pl.kernel,
        mesh=_mesh(),
        compiler_params=_sc_params,
        out_type=jax.ShapeDtypeStruct((2, NPAD, D), jnp.float32),
        scratch_types=[
            pltpu.VMEM((nch, CHUNK), jnp.int32),
            pltpu.VMEM((nch, CHUNK), jnp.int32),
            pltpu.VMEM((nbuf, kb, CHUNK, D), jnp.float32),
            pltpu.VMEM_SHARED((NPAD, D), jnp.float32),
            [pltpu.SemaphoreType.DMA] * 8,
            [pltpu.SemaphoreType.DMA] * 8,
        ],
    )
    def agg_kernel(y_hbm, edge_hbm, spad_hbm, dpad_hbm, out_hbm,
                   src_v, dst_v, rows_v, acc_sh, gsems, ssems):
        c = lax.axis_index("c")
        s = lax.axis_index("s")
        wid = s * 2 + c
        _load_idx(edge_hbm.at[0], spad_hbm, src_v, wid, nch, nreal)
        _load_idx(edge_hbm.at[1], dpad_hbm, dst_v, wid, nch, nreal)
        pltpu.sync_copy(y_hbm.at[pl.ds(s * ROWS_PT, ROWS_PT)],
                        acc_sh.at[pl.ds(s * ROWS_PT, ROWS_PT)])
        plsc.subcore_barrier()

        def gather(g, b):
            for k in range(kb):
                pltpu.async_copy(y_hbm.at[src_v.at[g * kb + k]],
                                 rows_v.at[b, k], gsems[b])

        def gather_wait(g, b):
            for k in range(kb):
                pltpu.make_async_copy(y_hbm.at[src_v.at[g * kb + k]],
                                      rows_v.at[b, k], gsems[b]).wait()

        def scatter(g, b):
            for k in range(kb):
                pltpu.async_copy(rows_v.at[b, k],
                                 acc_sh.at[dst_v.at[g * kb + k]],
                                 ssems[b], add=True)

        def scatter_wait(g, b):
            for k in range(kb):
                pltpu.make_async_copy(rows_v.at[b, k],
                                      acc_sh.at[dst_v.at[g * kb + k]],
                                      ssems[b]).wait()

        for g0 in range(lookahead):
            gather(g0, g0)

        def body(i, _):
            for bi in range(nbuf):
                g = i * nbuf + bi
                gather_wait(g, bi)
                scatter(g, bi)
                nb = (bi + lookahead) % nbuf
                lag = nbuf - lookahead

                @pl.when(g >= lag)
                def _drain():
                    scatter_wait(g - lag, nb)

                @pl.when(g + lookahead < nsup)
                def _fire():
                    gather(g + lookahead, nb)
            return _

        lax.fori_loop(0, nsup // nbuf, body, None)
        for gt in range(nsup - (nbuf - lookahead), nsup):
            scatter_wait(gt, gt % nbuf)
        plsc.subcore_barrier()
        pltpu.sync_copy(acc_sh.at[pl.ds(s * ROWS_PT, ROWS_PT)],
                        out_hbm.at[c, pl.ds(s * ROWS_PT, ROWS_PT)])

    return agg_kernel


_NB = 1024
_NPAIR = NPAD // 2
_NPR = N_NODES // 2


def _pair_cols(W):
    k, n = W.shape
    z = jnp.zeros((k, n), W.dtype)
    return jnp.concatenate(
        [jnp.concatenate([W, z], axis=1),
         jnp.concatenate([z, W], axis=1)], axis=0)


def _tc_first(Xp, disp, W1b):
    def body(x_ref, m_ref, w_ref, y_ref):
        y_ref[...] = m_ref[...] * jnp.dot(x_ref[...], w_ref[...],
                                          preferred_element_type=jnp.float32)

    return pl.pallas_call(
        body,
        grid=(_NPAIR // _NB,),
        in_specs=[
            pl.BlockSpec((_NB, 256), lambda i: (i, 0)),
            pl.BlockSpec((_NB, 128), lambda i: (i, 0)),
            pl.BlockSpec((256, 128), lambda i: (0, 0)),
        ],
        out_specs=pl.BlockSpec((_NB, 128), lambda i: (i, 0)),
        out_shape=jax.ShapeDtypeStruct((_NPAIR, 128), jnp.float32),
    )(Xp, disp, W1b)


def _tc_mid(S_pair, Y, disp, c3p, W2b, b1p):
    def body(s_ref, y_ref, m_ref, c_ref, w_ref, b_ref, o_ref):
        tot = s_ref[0] + s_ref[1] - y_ref[...]
        h = jnp.maximum(tot * c_ref[...] + b_ref[...], 0.0)
        o_ref[...] = m_ref[...] * jnp.dot(h, w_ref[...],
                                          preferred_element_type=jnp.float32)

    return pl.pallas_call(
        body,
        grid=(_NPAIR // _NB,),
        in_specs=[
            pl.BlockSpec((2, _NB, 128), lambda i: (0, i, 0)),
            pl.BlockSpec((_NB, 128), lambda i: (i, 0)),
            pl.BlockSpec((_NB, 128), lambda i: (i, 0)),
            pl.BlockSpec((_NB, 128), lambda i: (i, 0)),
            pl.BlockSpec((128, 128), lambda i: (0, 0)),
            pl.BlockSpec((1, 128), lambda i: (0, 0)),
        ],
        out_specs=pl.BlockSpec((_NB, 128), lambda i: (i, 0)),
        out_shape=jax.ShapeDtypeStruct((_NPAIR, 128), jnp.float32),
    )(S_pair, Y, disp, c3p, W2b, b1p)


def _tc_last(S_pair, Y, c3p, b2p, Wcb, bcp):
    n2 = Wcb.shape[1]
    nb = 1000

    def body(s_ref, y_ref, c_ref, b2_ref, w_ref, bc_ref, o_ref):
        tot = s_ref[0] + s_ref[1] - y_ref[...]
        h = jnp.maximum(tot * c_ref[...] + b2_ref[...], 0.0)
        o_ref[...] = jnp.dot(h, w_ref[...],
                             preferred_element_type=jnp.float32) + bc_ref[...]

    return pl.pallas_call(
        body,
        grid=(_NPR // nb,),
        in_specs=[
            pl.BlockSpec((2, nb, 128), lambda i: (0, i, 0)),
            pl.BlockSpec((nb, 128), lambda i: (i, 0)),
            pl.BlockSpec((nb, 128), lambda i: (i, 0)),
            pl.BlockSpec((1, 128), lambda i: (0, 0)),
            pl.BlockSpec((128, n2), lambda i: (0, 0)),
            pl.BlockSpec((1, n2), lambda i: (0, 0)),
        ],
        out_specs=pl.BlockSpec((nb, n2), lambda i: (i, 0)),
        out_shape=jax.ShapeDtypeStruct((_NPR, n2), jnp.float32),
    )(S_pair, Y, c3p, b2p, Wcb, bcp)


def kernel(X, edge_index, W1, b1, W2, b2, Wc, bc):
    edges = edge_index.astype(jnp.int32)
    n_edges = edges.shape[1]
    if n_edges % CHUNK:
        extra = CHUNK - n_edges % CHUNK
        tail = jnp.stack([jnp.zeros((extra,), jnp.int32),
                          jnp.full((extra,), N_NODES, jnp.int32)])
        edges = jnp.concatenate([edges, tail], axis=1)
        n_edges += extra
    nreal = n_edges // CHUNK
    nch = -(-nreal // NW)
    nch = -(-nch // 8) * 8
    npadr = NW * nch - nreal

    edge3d = edges.reshape(2, nreal, CHUNK)
    rng = jnp.arange(npadr * CHUNK, dtype=jnp.int32)
    spad = (rng % N_NODES).reshape(npadr, CHUNK)
    dpad = (N_NODES + rng % (NPAD - N_NODES)).reshape(npadr, CHUNK)
    ones_init = jnp.ones((NPAD,), jnp.float32)

    deg_pair = _make_deg_kernel(nch, nreal)(edge3d, dpad, ones_init)
    deg = jnp.maximum(deg_pair[0] + deg_pair[1] - 1.0, 1.0)
    dis = lax.rsqrt(deg)
    disp = jnp.broadcast_to(dis[:, None], (NPAD, D)).reshape(_NPAIR, 128)
    c3p = jnp.broadcast_to((dis / deg)[:, None], (NPAD, D)).reshape(_NPAIR, 128)

    agg = _make_agg_kernel(nch, nreal)
    W1b = _pair_cols(W1)
    W2b = _pair_cols(W2)
    Wcb = _pair_cols(Wc)
    b1p = jnp.concatenate([b1, b1]).reshape(1, 128)
    b2p = jnp.concatenate([b2, b2]).reshape(1, 128)
    bcp = jnp.concatenate([bc, bc]).reshape(1, -1)
    Xp = X.reshape(N_NODES // 2, 256)

    Y1 = _tc_first(Xp, disp, W1b)
    S1 = agg(Y1.reshape(NPAD, D), edge3d, spad, dpad)
    S1p = S1.reshape(2, _NPAIR, 128)
    Y2 = _tc_mid(S1p, Y1, disp, c3p, W2b, b1p)
    S2 = agg(Y2.reshape(NPAD, D), edge3d, spad, dpad)
    S2p = S2.reshape(2, _NPAIR, 128)
    lp = _tc_last(S2p, Y2, c3p, b2p, Wcb, bcp)
    return lp.reshape(N_NODES, Wc.shape[1])


# speedup vs baseline: 1.0298x; 1.0298x over previous
import functools

import jax
import jax.numpy as jnp
from jax import lax
from jax.experimental import pallas as pl
from jax.experimental.pallas import tpu as pltpu
from jax.experimental.pallas import tpu_sc as plsc

N_NODES = 10000
NPAD = 10240
D = 64
NW = 32
CHUNK = 128
ROWS_PT = NPAD // 16

_mesh = functools.partial(
    plsc.VectorSubcoreMesh, core_axis_name="c", subcore_axis_name="s")
_sc_params = pltpu.CompilerParams(use_tc_tiling_on_sc=False)


def _load_idx(real_hbm, pad_hbm, idx_v, wid, nch, nreal):
    npadr = NW * nch - nreal
    nlast = nch - npadr
    if npadr == 0:
        pltpu.sync_copy(real_hbm.at[pl.ds(wid * nch, nch)], idx_v)
        return

    @pl.when(wid < NW - 1)
    def _full():
        pltpu.sync_copy(real_hbm.at[pl.ds(wid * nch, nch)], idx_v)

    @pl.when(wid == NW - 1)
    def _split():
        pltpu.sync_copy(real_hbm.at[pl.ds(nreal - nlast, nlast)],
                        idx_v.at[pl.ds(0, nlast)])
        pltpu.sync_copy(pad_hbm, idx_v.at[pl.ds(nlast, npadr)])


def _make_deg_kernel(nch, nreal):
    @functools.partial(
        pl.kernel,
        mesh=_mesh(),
        compiler_params=_sc_params,
        out_type=jax.ShapeDtypeStruct((2, NPAD), jnp.float32),
        scratch_types=[
            pltpu.VMEM((nch, CHUNK), jnp.int32),
            pltpu.VMEM((CHUNK,), jnp.float32),
            pltpu.VMEM_SHARED((NPAD,), jnp.float32),
            [pltpu.SemaphoreType.DMA] * 4,
        ],
    )
    def deg_kernel(edge_hbm, dpad_hbm, init_hbm, out_hbm,
                   dst_v, ones_v, acc_sh, ssems):
        c = lax.axis_index("c")
        s = lax.axis_index("s")
        wid = s * 2 + c
        _load_idx(edge_hbm.at[1], dpad_hbm, dst_v, wid, nch, nreal)
        pltpu.sync_copy(init_hbm.at[pl.ds(0, CHUNK)], ones_v)
        pltpu.sync_copy(init_hbm.at[pl.ds(s * ROWS_PT, ROWS_PT)],
                        acc_sh.at[pl.ds(s * ROWS_PT, ROWS_PT)])
        plsc.subcore_barrier()

        def body(i, _):
            for b in range(4):
                j = i * 4 + b

                @pl.when(j >= 4)
                def _drain():
                    pltpu.make_async_copy(
                        ones_v, acc_sh.at[dst_v.at[j - 4]], ssems[b]).wait()

                pltpu.async_copy(ones_v, acc_sh.at[dst_v.at[j]],
                                 ssems[b], add=True)
            return _

        lax.fori_loop(0, nch // 4, body, None)
        for b in range(4):
            pltpu.make_async_copy(ones_v,
                                  acc_sh.at[dst_v.at[nch - 4 + b]],
                                  ssems[b]).wait()
        plsc.subcore_barrier()
        pltpu.sync_copy(acc_sh.at[pl.ds(s * ROWS_PT, ROWS_PT)],
                        out_hbm.at[c, pl.ds(s * ROWS_PT, ROWS_PT)])

    return deg_kernel


def _make_agg_kernel(nch, nreal):
    kb = 1
    nbuf = 8
    lookahead = 4
    nsup = nch // kb

    @functools.partial(
        pl.kernel,
        mesh=_mesh(),
        compiler_params=_sc_params,
        out_type=jax.ShapeDtypeStruct((2, NPAD, D), jnp.float32),
        scratch_types=[
            pltpu.VMEM((nch, CHUNK), jnp.int32),
            pltpu.VMEM((nch, CHUNK), jnp.int32),
            pltpu.VMEM((nbuf, kb, CHUNK, D), jnp.float32),
            pltpu.VMEM_SHARED((NPAD, D), jnp.float32),
            [pltpu.SemaphoreType.DMA] * 8,
            [pltpu.SemaphoreType.DMA] * 8,
        ],
    )
    def agg_kernel(y_hbm, edge_hbm, spad_hbm, dpad_hbm, out_hbm,
                   src_v, dst_v, rows_v, acc_sh, gsems, ssems):
        c = lax.axis_index("c")
        s = lax.axis_index("s")
        wid = s * 2 + c
        _load_idx(edge_hbm.at[0], spad_hbm, src_v, wid, nch, nreal)
        _load_idx(edge_hbm.at[1], dpad_hbm, dst_v, wid, nch, nreal)

        def gather(g, b):
            for k in range(kb):
                pltpu.async_copy(y_hbm.at[src_v.at[g * kb + k]],
                                 rows_v.at[b, k], gsems[b])

        def gather_wait(g, b):
            for k in range(kb):
                pltpu.make_async_copy(y_hbm.at[src_v.at[g * kb + k]],
                                      rows_v.at[b, k], gsems[b]).wait()

        def scatter(g, b):
            for k in range(kb):
                pltpu.async_copy(rows_v.at[b, k],
                                 acc_sh.at[dst_v.at[g * kb + k]],
                                 ssems[b], add=True)

        def scatter_wait(g, b):
            for k in range(kb):
                pltpu.make_async_copy(rows_v.at[b, k],
                                      acc_sh.at[dst_v.at[g * kb + k]],
                                      ssems[b]).wait()

        for g0 in range(lookahead):
            gather(g0, g0)
        pltpu.sync_copy(y_hbm.at[pl.ds(s * ROWS_PT, ROWS_PT)],
                        acc_sh.at[pl.ds(s * ROWS_PT, ROWS_PT)])
        plsc.subcore_barrier()

        def body(i, _):
            for bi in range(nbuf):
                g = i * nbuf + bi
                gather_wait(g, bi)
                scatter(g, bi)
                nb = (bi + lookahead) % nbuf
                lag = nbuf - lookahead

                @pl.when(g >= lag)
                def _drain():
                    scatter_wait(g - lag, nb)

                @pl.when(g + lookahead < nsup)
                def _fire():
                    gather(g + lookahead, nb)
            return _

        lax.fori_loop(0, nsup // nbuf, body, None)
        for gt in range(nsup - (nbuf - lookahead), nsup):
            scatter_wait(gt, gt % nbuf)
        plsc.subcore_barrier()
        pltpu.sync_copy(acc_sh.at[pl.ds(s * ROWS_PT, ROWS_PT)],
                        out_hbm.at[c, pl.ds(s * ROWS_PT, ROWS_PT)])

    return agg_kernel


_NB = 1024
_NPAIR = NPAD // 2
_NPR = N_NODES // 2


def _pair_cols(W):
    k, n = W.shape
    z = jnp.zeros((k, n), W.dtype)
    return jnp.concatenate(
        [jnp.concatenate([W, z], axis=1),
         jnp.concatenate([z, W], axis=1)], axis=0)


def _tc_first(Xp, disp, W1b):
    def body(x_ref, m_ref, w_ref, y_ref):
        y_ref[...] = m_ref[...] * jnp.dot(x_ref[...], w_ref[...],
                                          preferred_element_type=jnp.float32)

    return pl.pallas_call(
        body,
        grid=(_NPAIR // _NB,),
        in_specs=[
            pl.BlockSpec((_NB, 256), lambda i: (i, 0)),
            pl.BlockSpec((_NB, 128), lambda i: (i, 0)),
            pl.BlockSpec((256, 128), lambda i: (0, 0)),
        ],
        out_specs=pl.BlockSpec((_NB, 128), lambda i: (i, 0)),
        out_shape=jax.ShapeDtypeStruct((_NPAIR, 128), jnp.float32),
    )(Xp, disp, W1b)


def _tc_mid(S_pair, Y, disp, c3p, W2b, b1p):
    def body(s_ref, y_ref, m_ref, c_ref, w_ref, b_ref, o_ref):
        tot = s_ref[0] + s_ref[1] - y_ref[...]
        h = jnp.maximum(tot * c_ref[...] + b_ref[...], 0.0)
        o_ref[...] = m_ref[...] * jnp.dot(h, w_ref[...],
                                          preferred_element_type=jnp.float32)

    return pl.pallas_call(
        body,
        grid=(_NPAIR // _NB,),
        in_specs=[
            pl.BlockSpec((2, _NB, 128), lambda i: (0, i, 0)),
            pl.BlockSpec((_NB, 128), lambda i: (i, 0)),
            pl.BlockSpec((_NB, 128), lambda i: (i, 0)),
            pl.BlockSpec((_NB, 128), lambda i: (i, 0)),
            pl.BlockSpec((128, 128), lambda i: (0, 0)),
            pl.BlockSpec((1, 128), lambda i: (0, 0)),
        ],
        out_specs=pl.BlockSpec((_NB, 128), lambda i: (i, 0)),
        out_shape=jax.ShapeDtypeStruct((_NPAIR, 128), jnp.float32),
    )(S_pair, Y, disp, c3p, W2b, b1p)


def _tc_last(S_pair, Y, c3p, b2p, Wcb, bcp):
    n2 = Wcb.shape[1]
    nb = 1000

    def body(s_ref, y_ref, c_ref, b2_ref, w_ref, bc_ref, o_ref):
        tot = s_ref[0] + s_ref[1] - y_ref[...]
        h = jnp.maximum(tot * c_ref[...] + b2_ref[...], 0.0)
        o_ref[...] = jnp.dot(h, w_ref[...],
                             preferred_element_type=jnp.float32) + bc_ref[...]

    return pl.pallas_call(
        body,
        grid=(_NPR // nb,),
        in_specs=[
            pl.BlockSpec((2, nb, 128), lambda i: (0, i, 0)),
            pl.BlockSpec((nb, 128), lambda i: (i, 0)),
            pl.BlockSpec((nb, 128), lambda i: (i, 0)),
            pl.BlockSpec((1, 128), lambda i: (0, 0)),
            pl.BlockSpec((128, n2), lambda i: (0, 0)),
            pl.BlockSpec((1, n2), lambda i: (0, 0)),
        ],
        out_specs=pl.BlockSpec((nb, n2), lambda i: (i, 0)),
        out_shape=jax.ShapeDtypeStruct((_NPR, n2), jnp.float32),
    )(S_pair, Y, c3p, b2p, Wcb, bcp)


def kernel(X, edge_index, W1, b1, W2, b2, Wc, bc):
    edges = edge_index.astype(jnp.int32)
    n_edges = edges.shape[1]
    if n_edges % CHUNK:
        extra = CHUNK - n_edges % CHUNK
        tail = jnp.stack([jnp.zeros((extra,), jnp.int32),
                          jnp.full((extra,), N_NODES, jnp.int32)])
        edges = jnp.concatenate([edges, tail], axis=1)
        n_edges += extra
    nreal = n_edges // CHUNK
    nch = -(-nreal // NW)
    nch = -(-nch // 8) * 8
    npadr = NW * nch - nreal

    edge3d = edges.reshape(2, nreal, CHUNK)
    rng = jnp.arange(npadr * CHUNK, dtype=jnp.int32)
    spad = (rng % N_NODES).reshape(npadr, CHUNK)
    dpad = (N_NODES + rng % (NPAD - N_NODES)).reshape(npadr, CHUNK)
    ones_init = jnp.ones((NPAD,), jnp.float32)

    deg_pair = _make_deg_kernel(nch, nreal)(edge3d, dpad, ones_init)
    deg = jnp.maximum(deg_pair[0] + deg_pair[1] - 1.0, 1.0)
    dis = lax.rsqrt(deg)
    disp = jnp.broadcast_to(dis[:, None], (NPAD, D)).reshape(_NPAIR, 128)
    c3p = jnp.broadcast_to((dis / deg)[:, None], (NPAD, D)).reshape(_NPAIR, 128)

    agg = _make_agg_kernel(nch, nreal)
    W1b = _pair_cols(W1)
    W2b = _pair_cols(W2)
    Wcb = _pair_cols(Wc)
    b1p = jnp.concatenate([b1, b1]).reshape(1, 128)
    b2p = jnp.concatenate([b2, b2]).reshape(1, 128)
    bcp = jnp.concatenate([bc, bc]).reshape(1, -1)
    Xp = X.reshape(N_NODES // 2, 256)

    Y1 = _tc_first(Xp, disp, W1b)
    S1 = agg(Y1.reshape(NPAD, D), edge3d, spad, dpad)
    S1p = S1.reshape(2, _NPAIR, 128)
    Y2 = _tc_mid(S1p, Y1, disp, c3p, W2b, b1p)
    S2 = agg(Y2.reshape(NPAD, D), edge3d, spad, dpad)
    S2p = S2.reshape(2, _NPAIR, 128)
    lp = _tc_last(S2p, Y2, c3p, b2p, Wcb, bcp)
    return lp.reshape(N_NODES, Wc.shape[1])
